# Initial kernel scaffold; baseline (speedup 1.0000x reference)
import jax
import jax.numpy as jnp
from jax.experimental import pallas as pl


def kernel(x, edge_index, edge_mask, batch, W1, b1, W2, b2, W3, b3, Wl, bl):
    raise NotImplementedError("write your pallas kernel here")


# speedup vs baseline: 10.0895x; 10.0895x over previous
import functools

import jax
import jax.numpy as jnp
from jax import lax
from jax.experimental import pallas as pl
from jax.experimental.pallas import tpu as pltpu
from jax.experimental.pallas import tpu_sc as plsc

N = 10000
E = 320000
F = 128
G = 256

NC, NS, L = 2, 16, 16
NW = NC * NS
N_P = 10240
DUMMY = N_P - 8
C = 128
CHUNKS = -(-E // (NW * C))
EW = CHUNKS * C
E_P = NW * EW
RB = 1024
GRID = N_P // RB
RS = N_P // NS

_MESH = dict(core_axis_name="c", subcore_axis_name="s", num_cores=NC,
             num_subcores=NS)
_HIGH = lax.Precision.HIGHEST


DW = F


def _degree_body(dst_hbm, msk_hbm, dpart_hbm, acc, vals, stage, dbuf, mbuf,
                 ibuf):
    cid = lax.axis_index("c")
    sid = lax.axis_index("s")
    wid = cid * NS + sid
    z16 = jnp.zeros((L,), jnp.float32)
    one16 = jnp.ones((L,), jnp.float32)

    def zero_vals(i, carry):
        for g in range(DW // L):
            vals[i, pl.ds(g * L, L)] = z16
        return carry

    lax.fori_loop(0, C, zero_vals, 0)
    for k in range(RS // C):
        pltpu.sync_copy(vals, acc.at[pl.ds(sid * RS + k * C, C)])
    plsc.subcore_barrier()

    base = wid * EW
    dmy16 = jnp.full((L,), DUMMY, jnp.int32)

    for p in range(3):
        def set_vals(i, carry, p=p):
            for g in range(DW // L):
                vals[i, pl.ds(g * L, L)] = one16 if g == p else z16
            return carry

        lax.fori_loop(0, C, set_vals, 0)

        def chunk_body(k, carry, p=p):
            off = base + k * C
            pltpu.sync_copy(dst_hbm.at[pl.ds(off, C)], dbuf)
            pltpu.sync_copy(msk_hbm.at[pl.ds(off, C)], mbuf)
            for j in range(C // L):
                d = dbuf[pl.ds(j * L, L)]
                m = mbuf[pl.ds(j * L, L)]
                keep = (m < 3) if p == 0 else (m == p)
                ibuf[pl.ds(j * L, L)] = jnp.where(keep, d, dmy16)
            pltpu.sync_copy(vals, acc.at[ibuf], add=True)
            return carry

        lax.fori_loop(0, CHUNKS, chunk_body, 0)

    plsc.subcore_barrier()
    for k in range(RS // C):
        pltpu.sync_copy(acc.at[pl.ds(sid * RS + k * C, C)], stage)
        pltpu.sync_copy(
            stage, dpart_hbm.at[pl.ds(cid * N_P + sid * RS + k * C, C)])


def _make_degree_kernel():
    return pl.kernel(
        _degree_body,
        out_type=jax.ShapeDtypeStruct((NC * N_P, DW), jnp.float32),
        mesh=plsc.VectorSubcoreMesh(**_MESH),
        scratch_types=[
            pltpu.VMEM_SHARED((N_P, DW), jnp.float32),
            pltpu.VMEM((C, DW), jnp.float32),
            pltpu.VMEM((C, DW), jnp.float32),
            pltpu.VMEM((C,), jnp.int32),
            pltpu.VMEM((C,), jnp.int32),
            pltpu.VMEM((C,), jnp.int32),
        ],
    )


def _spmm_body(sel, hs_hbm, src_hbm, dst_hbm, msk_hbm, tpart_hbm,
               acc, rows, stage, sbuf, dbuf, mbuf, sem):
    cid = lax.axis_index("c")
    sid = lax.axis_index("s")
    wid = cid * NS + sid
    z16 = jnp.zeros((L,), jnp.float32)

    def zero_rows(i, carry):
        for j in range(F // L):
            rows[i, pl.ds(j * L, L)] = z16
        return carry

    lax.fori_loop(0, C, zero_rows, 0)
    for k in range(RS // C):
        pltpu.sync_copy(rows, acc.at[pl.ds(sid * RS + k * C, C)])
    plsc.subcore_barrier()

    base = wid * EW
    dmy16 = jnp.full((L,), DUMMY, jnp.int32)

    def chunk_body(k, carry):
        off = base + k * C
        pltpu.sync_copy(src_hbm.at[pl.ds(off, C)], sbuf)
        pltpu.sync_copy(dst_hbm.at[pl.ds(off, C)], dbuf)
        pltpu.sync_copy(msk_hbm.at[pl.ds(off, C)], mbuf)
        pltpu.async_copy(hs_hbm.at[sbuf], rows, sem).wait()
        for j in range(C // L):
            d = dbuf[pl.ds(j * L, L)]
            m = mbuf[pl.ds(j * L, L)]
            keep = (m < 3) if sel < 0 else (m == sel)
            dbuf[pl.ds(j * L, L)] = jnp.where(keep, d, dmy16)
        pltpu.sync_copy(rows, acc.at[dbuf], add=True)
        return carry

    lax.fori_loop(0, CHUNKS, chunk_body, 0)
    plsc.subcore_barrier()
    for k in range(RS // C):
        pltpu.sync_copy(acc.at[pl.ds(sid * RS + k * C, C)], stage)
        pltpu.sync_copy(
            stage, tpart_hbm.at[pl.ds(cid * N_P + sid * RS + k * C, C)])


def _make_spmm_kernel(sel):
    return pl.kernel(
        functools.partial(_spmm_body, sel),
        out_type=jax.ShapeDtypeStruct((NC * N_P, F), jnp.float32),
        mesh=plsc.VectorSubcoreMesh(**_MESH),
        scratch_types=[
            pltpu.VMEM_SHARED((N_P, F), jnp.float32),
            pltpu.VMEM((C, F), jnp.float32),
            pltpu.VMEM((C, F), jnp.float32),
            pltpu.VMEM((C,), jnp.int32),
            pltpu.VMEM((C,), jnp.int32),
            pltpu.VMEM((C,), jnp.int32),
            pltpu.SemaphoreType.DMA,
        ],
    )


def _dinv(dp_ref, li):
    deg = dp_ref[0][:, li * L:li * L + 1] + dp_ref[1][:, li * L:li * L + 1]
    d = deg + 1.0
    y = lax.rsqrt(d)
    return y * (1.5 - 0.5 * d * y * y)


def _prep_body(x_ref, w1_ref, dp_ref, hs_ref):
    h = jnp.dot(x_ref[...], w1_ref[...], preferred_element_type=jnp.float32,
                precision=_HIGH)
    hs_ref[...] = h * _dinv(dp_ref, 0)


def _prep_call(x_p, W1, dpart):
    return pl.pallas_call(
        _prep_body,
        grid=(GRID,),
        in_specs=[
            pl.BlockSpec((RB, F), lambda i: (i, 0)),
            pl.BlockSpec((F, F), lambda i: (0, 0)),
            pl.BlockSpec((NC, RB, DW), lambda i: (0, i, 0)),
        ],
        out_specs=pl.BlockSpec((RB, F), lambda i: (i, 0)),
        out_shape=jax.ShapeDtypeStruct((N_P, F), jnp.float32),
    )(x_p, W1, dpart)


def _fuse_body(li, tp_ref, hs_ref, dp_ref, b_ref, w_ref, out_ref):
    t = tp_ref[0] + tp_ref[1] + hs_ref[...]
    agg = t * _dinv(dp_ref, li) + b_ref[...]
    h = jnp.maximum(agg, 0.0)
    hn = jnp.dot(h, w_ref[...], preferred_element_type=jnp.float32,
                 precision=_HIGH)
    out_ref[...] = hn * _dinv(dp_ref, li + 1)


def _fuse_call(li, tpart, hs, dpart, b_row, W_next):
    return pl.pallas_call(
        functools.partial(_fuse_body, li),
        grid=(GRID,),
        in_specs=[
            pl.BlockSpec((NC, RB, F), lambda i: (0, i, 0)),
            pl.BlockSpec((RB, F), lambda i: (i, 0)),
            pl.BlockSpec((NC, RB, DW), lambda i: (0, i, 0)),
            pl.BlockSpec((1, F), lambda i: (0, 0)),
            pl.BlockSpec((F, F), lambda i: (0, 0)),
        ],
        out_specs=pl.BlockSpec((RB, F), lambda i: (i, 0)),
        out_shape=jax.ShapeDtypeStruct((N_P, F), jnp.float32),
    )(tpart, hs, dpart, b_row, W_next)


def _final_body(tp_ref, hs_ref, dp_ref, b_ref, batch_ref, wl_ref, bl_ref,
                out_ref, sums, cnts):
    i = pl.program_id(0)

    @pl.when(i == 0)
    def _init():
        sums[...] = jnp.zeros_like(sums)
        cnts[...] = jnp.zeros_like(cnts)

    t = tp_ref[0] + tp_ref[1] + hs_ref[...]
    agg = t * _dinv(dp_ref, 2) + b_ref[...]
    b = batch_ref[...].reshape(RB)
    onehot = (lax.broadcasted_iota(jnp.int32, (G, RB), 0)
              == b[None, :]).astype(jnp.float32)
    sums[...] += jnp.dot(onehot, agg, preferred_element_type=jnp.float32,
                         precision=_HIGH)
    cnt = jnp.sum(onehot, axis=1)
    cnts[...] += jnp.broadcast_to(cnt[:, None], (G, F))

    @pl.when(i == GRID - 1)
    def _head():
        pooled = sums[...] / jnp.maximum(cnts[...], 1.0)
        out_ref[...] = jnp.dot(pooled, wl_ref[...],
                               preferred_element_type=jnp.float32,
                               precision=_HIGH) + bl_ref[...]


def _final_call(tpart, hs, dpart, b_row, batch3, wl_pad, bl_row):
    return pl.pallas_call(
        _final_body,
        grid=(GRID,),
        in_specs=[
            pl.BlockSpec((NC, RB, F), lambda i: (0, i, 0)),
            pl.BlockSpec((RB, F), lambda i: (i, 0)),
            pl.BlockSpec((NC, RB, DW), lambda i: (0, i, 0)),
            pl.BlockSpec((1, F), lambda i: (0, 0)),
            pl.BlockSpec((1, 1, RB), lambda i: (i, 0, 0)),
            pl.BlockSpec((F, F), lambda i: (0, 0)),
            pl.BlockSpec((1, F), lambda i: (0, 0)),
        ],
        out_specs=pl.BlockSpec((G, F), lambda i: (0, 0)),
        out_shape=jax.ShapeDtypeStruct((G, F), jnp.float32),
        scratch_shapes=[
            pltpu.VMEM((G, F), jnp.float32),
            pltpu.VMEM((G, F), jnp.float32),
        ],
    )(tpart, hs, dpart, b_row, batch3, wl_pad, bl_row)


def kernel(x, edge_index, edge_mask, batch, W1, b1, W2, b2, W3, b3, Wl, bl):
    i32, f32 = jnp.int32, jnp.float32
    src = edge_index[0]
    dst = edge_index[1]
    pe = E_P - E
    src_p = jnp.concatenate([src, jnp.zeros((pe,), i32)])
    dst_p = jnp.concatenate([dst, jnp.zeros((pe,), i32)])
    msk_p = jnp.concatenate([edge_mask, jnp.full((pe,), 3, i32)])
    x_p = jnp.concatenate([x, jnp.zeros((N_P - N, F), f32)], axis=0)
    batch3 = jnp.concatenate([batch, jnp.full((N_P - N,), G, i32)]
                             ).reshape(GRID, 1, RB)
    wl_pad = jnp.pad(Wl, ((0, 0), (0, F - 1)))
    bl_row = jnp.broadcast_to(bl, (1, F))
    b1r = b1.reshape(1, F)
    b2r = b2.reshape(1, F)
    b3r = b3.reshape(1, F)

    dpart = _make_degree_kernel()(dst_p, msk_p).reshape(NC, N_P, DW)
    hs1 = _prep_call(x_p, W1, dpart)
    t1 = _make_spmm_kernel(-1)(hs1, src_p, dst_p, msk_p).reshape(NC, N_P, F)
    hs2 = _fuse_call(0, t1, hs1, dpart, b1r, W2)
    t2 = _make_spmm_kernel(1)(hs2, src_p, dst_p, msk_p).reshape(NC, N_P, F)
    hs3 = _fuse_call(1, t2, hs2, dpart, b2r, W3)
    t3 = _make_spmm_kernel(2)(hs3, src_p, dst_p, msk_p).reshape(NC, N_P, F)
    out = _final_call(t3, hs3, dpart, b3r, batch3, wl_pad, bl_row)
    return out[:, :1]


# speedup vs baseline: 10.1336x; 1.0044x over previous
import functools

import jax
import jax.numpy as jnp
from jax import lax
from jax.experimental import pallas as pl
from jax.experimental.pallas import tpu as pltpu
from jax.experimental.pallas import tpu_sc as plsc

N = 10000
E = 320000
F = 128
G = 256

NC, NS, L = 2, 16, 16
NW = NC * NS
N_P = 10240
DUMMY = N_P - 8
C = 128
CHUNKS = -(-E // (NW * C))
EW = CHUNKS * C
E_P = NW * EW
RB = 1024
GRID = N_P // RB
RS = N_P // NS

_MESH = dict(core_axis_name="c", subcore_axis_name="s", num_cores=NC,
             num_subcores=NS)
_PREC = None


DW = F


def _degree_body(dst_hbm, msk_hbm, dpart_hbm, acc, vals, stage, dbuf, mbuf,
                 ibuf):
    cid = lax.axis_index("c")
    sid = lax.axis_index("s")
    wid = cid * NS + sid
    z16 = jnp.zeros((L,), jnp.float32)
    one16 = jnp.ones((L,), jnp.float32)

    def zero_vals(i, carry):
        for g in range(DW // L):
            vals[i, pl.ds(g * L, L)] = z16
        return carry

    lax.fori_loop(0, C, zero_vals, 0)
    for k in range(RS // C):
        pltpu.sync_copy(vals, acc.at[pl.ds(sid * RS + k * C, C)])
    plsc.subcore_barrier()

    base = wid * EW
    dmy16 = jnp.full((L,), DUMMY, jnp.int32)

    for p in range(3):
        def set_vals(i, carry, p=p):
            for g in range(DW // L):
                vals[i, pl.ds(g * L, L)] = one16 if g == p else z16
            return carry

        lax.fori_loop(0, C, set_vals, 0)

        def chunk_body(k, carry, p=p):
            off = base + k * C
            pltpu.sync_copy(dst_hbm.at[pl.ds(off, C)], dbuf)
            pltpu.sync_copy(msk_hbm.at[pl.ds(off, C)], mbuf)
            for j in range(C // L):
                d = dbuf[pl.ds(j * L, L)]
                m = mbuf[pl.ds(j * L, L)]
                keep = (m < 3) if p == 0 else (m == p)
                ibuf[pl.ds(j * L, L)] = jnp.where(keep, d, dmy16)
            pltpu.sync_copy(vals, acc.at[ibuf], add=True)
            return carry

        lax.fori_loop(0, CHUNKS, chunk_body, 0)

    plsc.subcore_barrier()
    for k in range(RS // C):
        pltpu.sync_copy(acc.at[pl.ds(sid * RS + k * C, C)], stage)
        pltpu.sync_copy(
            stage, dpart_hbm.at[pl.ds(cid * N_P + sid * RS + k * C, C)])


def _make_degree_kernel():
    return pl.kernel(
        _degree_body,
        out_type=jax.ShapeDtypeStruct((NC * N_P, DW), jnp.float32),
        mesh=plsc.VectorSubcoreMesh(**_MESH),
        scratch_types=[
            pltpu.VMEM_SHARED((N_P, DW), jnp.float32),
            pltpu.VMEM((C, DW), jnp.float32),
            pltpu.VMEM((C, DW), jnp.float32),
            pltpu.VMEM((C,), jnp.int32),
            pltpu.VMEM((C,), jnp.int32),
            pltpu.VMEM((C,), jnp.int32),
        ],
    )


def _spmm_body(sel, hs_hbm, src_hbm, dst_hbm, msk_hbm, tpart_hbm,
               acc, rows, stage, sbuf, dbuf, mbuf, sem):
    cid = lax.axis_index("c")
    sid = lax.axis_index("s")
    wid = cid * NS + sid
    z16 = jnp.zeros((L,), jnp.float32)

    def zero_rows(i, carry):
        for j in range(F // L):
            rows[i, pl.ds(j * L, L)] = z16
        return carry

    lax.fori_loop(0, C, zero_rows, 0)
    for k in range(RS // C):
        pltpu.sync_copy(rows, acc.at[pl.ds(sid * RS + k * C, C)])
    plsc.subcore_barrier()

    base = wid * EW
    dmy16 = jnp.full((L,), DUMMY, jnp.int32)

    def chunk_body(k, carry):
        off = base + k * C
        pltpu.sync_copy(src_hbm.at[pl.ds(off, C)], sbuf)
        pltpu.sync_copy(dst_hbm.at[pl.ds(off, C)], dbuf)
        pltpu.sync_copy(msk_hbm.at[pl.ds(off, C)], mbuf)
        pltpu.async_copy(hs_hbm.at[sbuf], rows, sem).wait()
        for j in range(C // L):
            d = dbuf[pl.ds(j * L, L)]
            m = mbuf[pl.ds(j * L, L)]
            keep = (m < 3) if sel < 0 else (m == sel)
            dbuf[pl.ds(j * L, L)] = jnp.where(keep, d, dmy16)
        pltpu.sync_copy(rows, acc.at[dbuf], add=True)
        return carry

    lax.fori_loop(0, CHUNKS, chunk_body, 0)
    plsc.subcore_barrier()
    for k in range(RS // C):
        pltpu.sync_copy(acc.at[pl.ds(sid * RS + k * C, C)], stage)
        pltpu.sync_copy(
            stage, tpart_hbm.at[pl.ds(cid * N_P + sid * RS + k * C, C)])


def _make_spmm_kernel(sel):
    return pl.kernel(
        functools.partial(_spmm_body, sel),
        out_type=jax.ShapeDtypeStruct((NC * N_P, F), jnp.float32),
        mesh=plsc.VectorSubcoreMesh(**_MESH),
        scratch_types=[
            pltpu.VMEM_SHARED((N_P, F), jnp.float32),
            pltpu.VMEM((C, F), jnp.float32),
            pltpu.VMEM((C, F), jnp.float32),
            pltpu.VMEM((C,), jnp.int32),
            pltpu.VMEM((C,), jnp.int32),
            pltpu.VMEM((C,), jnp.int32),
            pltpu.SemaphoreType.DMA,
        ],
    )


def _dinv(dp_ref, li):
    deg = dp_ref[0][:, li * L:li * L + 1] + dp_ref[1][:, li * L:li * L + 1]
    d = deg + 1.0
    y = lax.rsqrt(d)
    return y * (1.5 - 0.5 * d * y * y)


def _prep_body(x_ref, w1_ref, dp_ref, hs_ref):
    h = jnp.dot(x_ref[...], w1_ref[...], preferred_element_type=jnp.float32,
                precision=_PREC)
    hs_ref[...] = h * _dinv(dp_ref, 0)


def _prep_call(x_p, W1, dpart):
    return pl.pallas_call(
        _prep_body,
        grid=(GRID,),
        in_specs=[
            pl.BlockSpec((RB, F), lambda i: (i, 0)),
            pl.BlockSpec((F, F), lambda i: (0, 0)),
            pl.BlockSpec((NC, RB, DW), lambda i: (0, i, 0)),
        ],
        out_specs=pl.BlockSpec((RB, F), lambda i: (i, 0)),
        out_shape=jax.ShapeDtypeStruct((N_P, F), jnp.float32),
    )(x_p, W1, dpart)


def _fuse_body(li, tp_ref, hs_ref, dp_ref, b_ref, w_ref, out_ref):
    t = tp_ref[0] + tp_ref[1] + hs_ref[...]
    agg = t * _dinv(dp_ref, li) + b_ref[...]
    h = jnp.maximum(agg, 0.0)
    hn = jnp.dot(h, w_ref[...], preferred_element_type=jnp.float32,
                 precision=_PREC)
    out_ref[...] = hn * _dinv(dp_ref, li + 1)


def _fuse_call(li, tpart, hs, dpart, b_row, W_next):
    return pl.pallas_call(
        functools.partial(_fuse_body, li),
        grid=(GRID,),
        in_specs=[
            pl.BlockSpec((NC, RB, F), lambda i: (0, i, 0)),
            pl.BlockSpec((RB, F), lambda i: (i, 0)),
            pl.BlockSpec((NC, RB, DW), lambda i: (0, i, 0)),
            pl.BlockSpec((1, F), lambda i: (0, 0)),
            pl.BlockSpec((F, F), lambda i: (0, 0)),
        ],
        out_specs=pl.BlockSpec((RB, F), lambda i: (i, 0)),
        out_shape=jax.ShapeDtypeStruct((N_P, F), jnp.float32),
    )(tpart, hs, dpart, b_row, W_next)


def _final_body(tp_ref, hs_ref, dp_ref, b_ref, batch_ref, wl_ref, bl_ref,
                out_ref, sums, cnts):
    i = pl.program_id(0)

    @pl.when(i == 0)
    def _init():
        sums[...] = jnp.zeros_like(sums)
        cnts[...] = jnp.zeros_like(cnts)

    t = tp_ref[0] + tp_ref[1] + hs_ref[...]
    agg = t * _dinv(dp_ref, 2) + b_ref[...]
    b = batch_ref[...].reshape(RB)
    onehot = (lax.broadcasted_iota(jnp.int32, (G, RB), 0)
              == b[None, :]).astype(jnp.float32)
    sums[...] += jnp.dot(onehot, agg, preferred_element_type=jnp.float32,
                         precision=_PREC)
    cnt = jnp.sum(onehot, axis=1)
    cnts[...] += jnp.broadcast_to(cnt[:, None], (G, F))

    @pl.when(i == GRID - 1)
    def _head():
        pooled = sums[...] / jnp.maximum(cnts[...], 1.0)
        out_ref[...] = jnp.dot(pooled, wl_ref[...],
                               preferred_element_type=jnp.float32,
                               precision=_PREC) + bl_ref[...]


def _final_call(tpart, hs, dpart, b_row, batch3, wl_pad, bl_row):
    return pl.pallas_call(
        _final_body,
        grid=(GRID,),
        in_specs=[
            pl.BlockSpec((NC, RB, F), lambda i: (0, i, 0)),
            pl.BlockSpec((RB, F), lambda i: (i, 0)),
            pl.BlockSpec((NC, RB, DW), lambda i: (0, i, 0)),
            pl.BlockSpec((1, F), lambda i: (0, 0)),
            pl.BlockSpec((1, 1, RB), lambda i: (i, 0, 0)),
            pl.BlockSpec((F, F), lambda i: (0, 0)),
            pl.BlockSpec((1, F), lambda i: (0, 0)),
        ],
        out_specs=pl.BlockSpec((G, F), lambda i: (0, 0)),
        out_shape=jax.ShapeDtypeStruct((G, F), jnp.float32),
        scratch_shapes=[
            pltpu.VMEM((G, F), jnp.float32),
            pltpu.VMEM((G, F), jnp.float32),
        ],
    )(tpart, hs, dpart, b_row, batch3, wl_pad, bl_row)


def kernel(x, edge_index, edge_mask, batch, W1, b1, W2, b2, W3, b3, Wl, bl):
    i32, f32 = jnp.int32, jnp.float32
    src = edge_index[0]
    dst = edge_index[1]
    pe = E_P - E
    src_p = jnp.concatenate([src, jnp.zeros((pe,), i32)])
    dst_p = jnp.concatenate([dst, jnp.zeros((pe,), i32)])
    msk_p = jnp.concatenate([edge_mask, jnp.full((pe,), 3, i32)])
    x_p = jnp.concatenate([x, jnp.zeros((N_P - N, F), f32)], axis=0)
    batch3 = jnp.concatenate([batch, jnp.full((N_P - N,), G, i32)]
                             ).reshape(GRID, 1, RB)
    wl_pad = jnp.pad(Wl, ((0, 0), (0, F - 1)))
    bl_row = jnp.broadcast_to(bl, (1, F))
    b1r = b1.reshape(1, F)
    b2r = b2.reshape(1, F)
    b3r = b3.reshape(1, F)

    dpart = _make_degree_kernel()(dst_p, msk_p).reshape(NC, N_P, DW)
    hs1 = _prep_call(x_p, W1, dpart)
    t1 = _make_spmm_kernel(-1)(hs1, src_p, dst_p, msk_p).reshape(NC, N_P, F)
    hs2 = _fuse_call(0, t1, hs1, dpart, b1r, W2)
    t2 = _make_spmm_kernel(1)(hs2, src_p, dst_p, msk_p).reshape(NC, N_P, F)
    hs3 = _fuse_call(1, t2, hs2, dpart, b2r, W3)
    t3 = _make_spmm_kernel(2)(hs3, src_p, dst_p, msk_p).reshape(NC, N_P, F)
    out = _final_call(t3, hs3, dpart, b3r, batch3, wl_pad, bl_row)
    return out[:, :1]
